# baseline (speedup 1.0000x reference)
import functools

import jax
import jax.numpy as jnp
from jax import lax
from jax.experimental import pallas as pl
from jax.experimental.pallas import tpu as pltpu
from jax.experimental.pallas import tpu_sc as plsc

B, T, D = 1, 2048, 768
E, K, FF = 8, 2, 3072
N = B * T
S = N * K
BLK = 128
PT = S + E * BLK
NB = PT // BLK

_NC, _NS = 2, 16
NW = _NC * _NS
L = 16
PW = 128


def _router_body(x_ref, gw_ref, gb_ref, pos_ref, prob_ref, be_ref, loss_ref):
    logits = jnp.dot(x_ref[...], gw_ref[...], preferred_element_type=jnp.float32)
    logits = logits + gb_ref[0, :][None, :]
    ii = lax.broadcasted_iota(jnp.int32, (N, E), 1)
    m1 = jnp.max(logits, axis=1, keepdims=True)
    i1 = jnp.min(jnp.where(logits == m1, ii, E), axis=1, keepdims=True)
    l2 = jnp.where(ii == i1, -jnp.inf, logits)
    m2 = jnp.max(l2, axis=1, keepdims=True)
    i2 = jnp.min(jnp.where(l2 == m2, ii, E), axis=1, keepdims=True)
    d = jnp.exp(m2 - m1)
    p1 = 1.0 / (1.0 + d)
    p2 = d / (1.0 + d)
    prob_ref[...] = jnp.concatenate([p1, p2], axis=1)

    oh = jnp.where(ii == i1, 1.0, 0.0) + jnp.where(ii == i2, 1.0, 0.0)
    r = lax.broadcasted_iota(jnp.int32, (N, N), 0)
    c = lax.broadcasted_iota(jnp.int32, (N, N), 1)
    lt = jnp.where(c < r, 1.0, 0.0).astype(jnp.bfloat16)
    csum = jnp.dot(lt, oh.astype(jnp.bfloat16),
                   preferred_element_type=jnp.float32)
    rank1 = jnp.zeros((N, 1), jnp.float32)
    rank2 = jnp.zeros((N, 1), jnp.float32)
    for e in range(E):
        ce = csum[:, e][:, None]
        rank1 = rank1 + jnp.where(i1 == e, ce, 0.0)
        rank2 = rank2 + jnp.where(i2 == e, ce, 0.0)

    counts = jnp.sum(oh, axis=0, keepdims=True)
    loss = jnp.sum((counts / (N * K) - 1.0 / E) ** 2)
    loss_ref[...] = loss.reshape(1, 1)

    padded = jnp.floor((counts + (BLK - 1)) / BLK) * BLK
    ee_r = lax.broadcasted_iota(jnp.int32, (E, E), 0)
    ee_c = lax.broadcasted_iota(jnp.int32, (E, E), 1)
    lt8 = jnp.where(ee_r < ee_c, 1.0, 0.0)
    poff = jnp.dot(padded, lt8, preferred_element_type=jnp.float32)
    poffB = jnp.broadcast_to(poff, (N, E))
    pos1 = jnp.sum(jnp.where(ii == i1, poffB, 0.0), axis=1, keepdims=True) + rank1
    pos2 = jnp.sum(jnp.where(ii == i2, poffB, 0.0), axis=1, keepdims=True) + rank2
    pos_ref[...] = jnp.concatenate([pos1, pos2], axis=1).astype(jnp.int32)

    cpad = poff + padded
    bs = lax.broadcasted_iota(jnp.int32, (NB, 1), 0).astype(jnp.float32) * BLK
    ge = jnp.where(bs >= jnp.broadcast_to(cpad, (NB, E)), 1, 0)
    be_ref[...] = jnp.minimum(jnp.sum(ge, axis=1, keepdims=True), E - 1
                              ).astype(jnp.int32)


def _router(x_flat, gate_w, gate_b):
    return pl.pallas_call(
        _router_body,
        out_shape=(
            jax.ShapeDtypeStruct((N, K), jnp.int32),
            jax.ShapeDtypeStruct((N, K), jnp.float32),
            jax.ShapeDtypeStruct((NB, 1), jnp.int32),
            jax.ShapeDtypeStruct((1, 1), jnp.float32),
        ),
        compiler_params=pltpu.CompilerParams(
            vmem_limit_bytes=100 * 1024 * 1024),
    )(x_flat, gate_w, gate_b.reshape(1, E))


_D_TOK = N // NW


def _dispatch_body(x_hbm, p0_hbm, p1_hbm, pr0_hbm, pr1_hbm,
                   xg_hbm, prows_hbm,
                   i0_v, i1_v, rows_v, q0_v, q1_v, sem):
    wid = lax.axis_index("s") * _NC + lax.axis_index("c")
    base = wid * _D_TOK
    a0 = pltpu.async_copy(p0_hbm.at[pl.ds(base, _D_TOK)], i0_v, sem)
    a1 = pltpu.async_copy(p1_hbm.at[pl.ds(base, _D_TOK)], i1_v, sem)
    a2 = pltpu.async_copy(x_hbm.at[pl.ds(base, _D_TOK)], rows_v, sem)
    a3 = pltpu.async_copy(pr0_hbm.at[pl.ds(base, _D_TOK)], q0_v, sem)
    a4 = pltpu.async_copy(pr1_hbm.at[pl.ds(base, _D_TOK)], q1_v, sem)
    a0.wait()
    a1.wait()
    a2.wait()
    a3.wait()
    a4.wait()
    c0 = pltpu.async_copy(rows_v, xg_hbm.at[i0_v], sem)
    c1 = pltpu.async_copy(rows_v, xg_hbm.at[i1_v], sem)
    c2 = pltpu.async_copy(q0_v, prows_hbm.at[i0_v], sem)
    c3 = pltpu.async_copy(q1_v, prows_hbm.at[i1_v], sem)
    c0.wait()
    c1.wait()
    c2.wait()
    c3.wait()


def _dispatch_sc(x_flat, pos0, pos1, pr0, pr1):
    return pl.kernel(
        _dispatch_body,
        mesh=plsc.VectorSubcoreMesh(core_axis_name="c", subcore_axis_name="s"),
        out_type=(
            jax.ShapeDtypeStruct((PT, D), jnp.float32),
            jax.ShapeDtypeStruct((PT, PW), jnp.float32),
        ),
        scratch_types=[
            pltpu.VMEM((_D_TOK,), jnp.int32),
            pltpu.VMEM((_D_TOK,), jnp.int32),
            pltpu.VMEM((_D_TOK, D), jnp.float32),
            pltpu.VMEM((_D_TOK, PW), jnp.float32),
            pltpu.VMEM((_D_TOK, PW), jnp.float32),
            pltpu.SemaphoreType.DMA,
        ],
    )(x_flat, pos0, pos1, pr0, pr1)


def _ffn_body(be_ref, rt_ref, x_ref, w1_ref, b1_ref, w2_ref, b2_ref,
              wp_ref, bp_ref, p_ref, o_ref, xb_s):
    b = pl.program_id(0)
    base = b * BLK

    def cp(i, _):
        xb_s[i, :] = x_ref[rt_ref[base + i], :]
        return 0

    lax.fori_loop(0, BLK, cp, 0)
    xb = xb_s[...]
    h1 = jnp.dot(xb, w1_ref[0], preferred_element_type=jnp.float32) + b1_ref[0]
    h2 = jnp.dot(xb, w2_ref[0], preferred_element_type=jnp.float32) + b2_ref[0]
    s = (h1 * jax.nn.sigmoid(h1)) * h2
    y = jnp.dot(s, wp_ref[0], preferred_element_type=jnp.float32) + bp_ref[0]
    o_ref[...] = y * p_ref[0, 0][:, None]


def _ffn(be, rows_token, x_flat, w1, b1, w2, b2, wp, bp, rows_prob):
    grid_spec = pltpu.PrefetchScalarGridSpec(
        num_scalar_prefetch=2,
        grid=(NB,),
        in_specs=[
            pl.BlockSpec((N, D), lambda b, be_r, rt_r: (0, 0)),
            pl.BlockSpec((1, D, FF), lambda b, be_r, rt_r: (be_r[b], 0, 0)),
            pl.BlockSpec((1, 1, FF), lambda b, be_r, rt_r: (be_r[b], 0, 0)),
            pl.BlockSpec((1, D, FF), lambda b, be_r, rt_r: (be_r[b], 0, 0)),
            pl.BlockSpec((1, 1, FF), lambda b, be_r, rt_r: (be_r[b], 0, 0)),
            pl.BlockSpec((1, FF, D), lambda b, be_r, rt_r: (be_r[b], 0, 0)),
            pl.BlockSpec((1, 1, D), lambda b, be_r, rt_r: (be_r[b], 0, 0)),
            pl.BlockSpec((1, 1, BLK), lambda b, be_r, rt_r: (b, 0, 0)),
        ],
        out_specs=pl.BlockSpec((BLK, D), lambda b, be_r, rt_r: (b, 0)),
        scratch_shapes=[pltpu.VMEM((BLK, D), jnp.float32)],
    )
    return pl.pallas_call(
        _ffn_body,
        grid_spec=grid_spec,
        out_shape=jax.ShapeDtypeStruct((PT, D), jnp.float32),
        compiler_params=pltpu.CompilerParams(
            vmem_limit_bytes=110 * 1024 * 1024),
    )(be, rows_token, x_flat, w1, b1.reshape(E, 1, FF), w2,
      b2.reshape(E, 1, FF), wp, bp.reshape(E, 1, D),
      rows_prob.reshape(NB, 1, BLK))


_C_ROWS = N // NW


def _combine_body(y_hbm, p0_hbm, p1_hbm, out_hbm,
                  i0_v, i1_v, b0_v, b1_v, sem):
    wid = lax.axis_index("s") * _NC + lax.axis_index("c")
    base = wid * _C_ROWS
    pltpu.sync_copy(p0_hbm.at[pl.ds(base, _C_ROWS)], i0_v)
    pltpu.sync_copy(p1_hbm.at[pl.ds(base, _C_ROWS)], i1_v)
    c0 = pltpu.async_copy(y_hbm.at[i0_v], b0_v, sem)
    c1 = pltpu.async_copy(y_hbm.at[i1_v], b1_v, sem)
    c0.wait()
    c1.wait()

    def row(i, _):
        for j in range(D // L):
            sl = pl.ds(j * L, L)
            b0_v[i, sl] = b0_v[i, sl] + b1_v[i, sl]
        return 0

    lax.fori_loop(0, _C_ROWS, row, 0)
    pltpu.sync_copy(b0_v, out_hbm.at[pl.ds(base, _C_ROWS)])


def _combine_sc(y, pos0, pos1):
    return pl.kernel(
        _combine_body,
        mesh=plsc.VectorSubcoreMesh(core_axis_name="c", subcore_axis_name="s"),
        out_type=jax.ShapeDtypeStruct((N, D), jnp.float32),
        scratch_types=[
            pltpu.VMEM((_C_ROWS,), jnp.int32),
            pltpu.VMEM((_C_ROWS,), jnp.int32),
            pltpu.VMEM((_C_ROWS, D), jnp.float32),
            pltpu.VMEM((_C_ROWS, D), jnp.float32),
            pltpu.SemaphoreType.DMA,
        ],
    )(y, pos0, pos1)


def kernel(x, gate_w, gate_b, w1, b1, w2, b2, wp, bp):
    x_flat = x.reshape(N, D)
    pos, probs, be, loss = _router(x_flat, gate_w, gate_b)
    posS = pos.reshape(S)
    rows_token = jnp.zeros((PT,), jnp.int32).at[posS].set(
        jnp.arange(S, dtype=jnp.int32) // K)
    rows_prob = jnp.zeros((PT,), jnp.float32).at[posS].set(probs.reshape(S))
    y = _ffn(be.reshape(NB), rows_token, x_flat, w1, b1, w2, b2, wp, bp,
             rows_prob)
    out_flat = _combine_sc(y, pos[:, 0], pos[:, 1])
    return out_flat.reshape(B, T, D), loss.reshape(())


# speedup vs baseline: 1.3438x; 1.3438x over previous
import functools

import jax
import jax.numpy as jnp
from jax import lax
from jax.experimental import pallas as pl
from jax.experimental.pallas import tpu as pltpu
from jax.experimental.pallas import tpu_sc as plsc

B, T, D = 1, 2048, 768
E, K, FF = 8, 2, 3072
N = B * T
S = N * K
BLK = 256
PT = S + E * BLK
NB = PT // BLK

_NC, _NS = 2, 16
NW = _NC * _NS
L = 16
PW = 128


def _router_body(x_ref, gw_ref, gb_ref, pos_ref, prob_ref, be_ref, loss_ref):
    logits = jnp.dot(x_ref[...], gw_ref[...], preferred_element_type=jnp.float32)
    logits = logits + gb_ref[0, :][None, :]
    ii = lax.broadcasted_iota(jnp.int32, (N, E), 1)
    m1 = jnp.max(logits, axis=1, keepdims=True)
    i1 = jnp.min(jnp.where(logits == m1, ii, E), axis=1, keepdims=True)
    l2 = jnp.where(ii == i1, -jnp.inf, logits)
    m2 = jnp.max(l2, axis=1, keepdims=True)
    i2 = jnp.min(jnp.where(l2 == m2, ii, E), axis=1, keepdims=True)
    d = jnp.exp(m2 - m1)
    p1 = 1.0 / (1.0 + d)
    p2 = d / (1.0 + d)
    prob_ref[...] = jnp.concatenate([p1, p2], axis=1)

    oh = jnp.where(ii == i1, 1.0, 0.0) + jnp.where(ii == i2, 1.0, 0.0)
    r = lax.broadcasted_iota(jnp.int32, (N, N), 0)
    c = lax.broadcasted_iota(jnp.int32, (N, N), 1)
    lt = jnp.where(c < r, 1.0, 0.0).astype(jnp.bfloat16)
    csum = jnp.dot(lt, oh.astype(jnp.bfloat16),
                   preferred_element_type=jnp.float32)
    rank1 = jnp.zeros((N, 1), jnp.float32)
    rank2 = jnp.zeros((N, 1), jnp.float32)
    for e in range(E):
        ce = csum[:, e][:, None]
        rank1 = rank1 + jnp.where(i1 == e, ce, 0.0)
        rank2 = rank2 + jnp.where(i2 == e, ce, 0.0)

    counts = jnp.sum(oh, axis=0, keepdims=True)
    loss = jnp.sum((counts / (N * K) - 1.0 / E) ** 2)
    loss_ref[...] = loss.reshape(1, 1)

    padded = jnp.floor((counts + (BLK - 1)) / BLK) * BLK
    ee_r = lax.broadcasted_iota(jnp.int32, (E, E), 0)
    ee_c = lax.broadcasted_iota(jnp.int32, (E, E), 1)
    lt8 = jnp.where(ee_r < ee_c, 1.0, 0.0)
    poff = jnp.dot(padded, lt8, preferred_element_type=jnp.float32)
    poffB = jnp.broadcast_to(poff, (N, E))
    pos1 = jnp.sum(jnp.where(ii == i1, poffB, 0.0), axis=1, keepdims=True) + rank1
    pos2 = jnp.sum(jnp.where(ii == i2, poffB, 0.0), axis=1, keepdims=True) + rank2
    pos_ref[...] = jnp.concatenate([pos1, pos2], axis=1).astype(jnp.int32)

    cpad = poff + padded
    bs = lax.broadcasted_iota(jnp.int32, (NB, 1), 0).astype(jnp.float32) * BLK
    ge = jnp.where(bs >= jnp.broadcast_to(cpad, (NB, E)), 1, 0)
    be_ref[...] = jnp.minimum(jnp.sum(ge, axis=1, keepdims=True), E - 1
                              ).astype(jnp.int32)


def _router(x_flat, gate_w, gate_b):
    return pl.pallas_call(
        _router_body,
        out_shape=(
            jax.ShapeDtypeStruct((N, K), jnp.int32),
            jax.ShapeDtypeStruct((N, K), jnp.float32),
            jax.ShapeDtypeStruct((NB, 1), jnp.int32),
            jax.ShapeDtypeStruct((1, 1), jnp.float32),
        ),
        compiler_params=pltpu.CompilerParams(
            vmem_limit_bytes=100 * 1024 * 1024),
    )(x_flat, gate_w, gate_b.reshape(1, E))


_D_TOK = N // NW


def _dispatch_body(x_hbm, p0_hbm, p1_hbm, pr0_hbm, pr1_hbm,
                   xg_hbm, prows_hbm,
                   i0_v, i1_v, rows_v, q0_v, q1_v, sem):
    wid = lax.axis_index("s") * _NC + lax.axis_index("c")
    base = wid * _D_TOK
    a0 = pltpu.async_copy(p0_hbm.at[pl.ds(base, _D_TOK)], i0_v, sem)
    a1 = pltpu.async_copy(p1_hbm.at[pl.ds(base, _D_TOK)], i1_v, sem)
    a2 = pltpu.async_copy(x_hbm.at[pl.ds(base, _D_TOK)], rows_v, sem)
    a3 = pltpu.async_copy(pr0_hbm.at[pl.ds(base, _D_TOK)], q0_v, sem)
    a4 = pltpu.async_copy(pr1_hbm.at[pl.ds(base, _D_TOK)], q1_v, sem)
    a0.wait()
    a1.wait()
    a2.wait()
    a3.wait()
    a4.wait()
    c0 = pltpu.async_copy(rows_v, xg_hbm.at[i0_v], sem)
    c1 = pltpu.async_copy(rows_v, xg_hbm.at[i1_v], sem)
    c2 = pltpu.async_copy(q0_v, prows_hbm.at[i0_v], sem)
    c3 = pltpu.async_copy(q1_v, prows_hbm.at[i1_v], sem)
    c0.wait()
    c1.wait()
    c2.wait()
    c3.wait()


def _dispatch_sc(x_flat, pos0, pos1, pr0, pr1):
    return pl.kernel(
        _dispatch_body,
        mesh=plsc.VectorSubcoreMesh(core_axis_name="c", subcore_axis_name="s"),
        out_type=(
            jax.ShapeDtypeStruct((PT, D), jnp.float32),
            jax.ShapeDtypeStruct((PT, PW), jnp.float32),
        ),
        scratch_types=[
            pltpu.VMEM((_D_TOK,), jnp.int32),
            pltpu.VMEM((_D_TOK,), jnp.int32),
            pltpu.VMEM((_D_TOK, D), jnp.float32),
            pltpu.VMEM((_D_TOK, PW), jnp.float32),
            pltpu.VMEM((_D_TOK, PW), jnp.float32),
            pltpu.SemaphoreType.DMA,
        ],
    )(x_flat, pos0, pos1, pr0, pr1)


def _ffn_body(be_ref, xg_ref, w1_ref, b1_ref, w2_ref, b2_ref, wp_ref, bp_ref,
              p_ref, o_ref):
    xb = xg_ref[...]
    h1 = jnp.dot(xb, w1_ref[0], preferred_element_type=jnp.float32) + b1_ref[0]
    h2 = jnp.dot(xb, w2_ref[0], preferred_element_type=jnp.float32) + b2_ref[0]
    s = (h1 * jax.nn.sigmoid(h1)) * h2
    y = jnp.dot(s, wp_ref[0], preferred_element_type=jnp.float32) + bp_ref[0]
    o_ref[...] = y * p_ref[:, 0:1]


def _ffn(be, xg, w1, b1, w2, b2, wp, bp, prows):
    grid_spec = pltpu.PrefetchScalarGridSpec(
        num_scalar_prefetch=1,
        grid=(NB,),
        in_specs=[
            pl.BlockSpec((BLK, D), lambda b, be_r: (b, 0)),
            pl.BlockSpec((1, D, FF), lambda b, be_r: (be_r[b], 0, 0)),
            pl.BlockSpec((1, 1, FF), lambda b, be_r: (be_r[b], 0, 0)),
            pl.BlockSpec((1, D, FF), lambda b, be_r: (be_r[b], 0, 0)),
            pl.BlockSpec((1, 1, FF), lambda b, be_r: (be_r[b], 0, 0)),
            pl.BlockSpec((1, FF, D), lambda b, be_r: (be_r[b], 0, 0)),
            pl.BlockSpec((1, 1, D), lambda b, be_r: (be_r[b], 0, 0)),
            pl.BlockSpec((BLK, PW), lambda b, be_r: (b, 0)),
        ],
        out_specs=pl.BlockSpec((BLK, D), lambda b, be_r: (b, 0)),
    )
    return pl.pallas_call(
        _ffn_body,
        grid_spec=grid_spec,
        out_shape=jax.ShapeDtypeStruct((PT, D), jnp.float32),
        compiler_params=pltpu.CompilerParams(
            vmem_limit_bytes=100 * 1024 * 1024),
    )(be, xg, w1, b1.reshape(E, 1, FF), w2, b2.reshape(E, 1, FF),
      wp, bp.reshape(E, 1, D), prows)


_C_ROWS = N // NW


def _combine_body(y_hbm, p0_hbm, p1_hbm, out_hbm,
                  i0_v, i1_v, b0_v, b1_v, sem):
    wid = lax.axis_index("s") * _NC + lax.axis_index("c")
    base = wid * _C_ROWS
    pltpu.sync_copy(p0_hbm.at[pl.ds(base, _C_ROWS)], i0_v)
    pltpu.sync_copy(p1_hbm.at[pl.ds(base, _C_ROWS)], i1_v)
    c0 = pltpu.async_copy(y_hbm.at[i0_v], b0_v, sem)
    c1 = pltpu.async_copy(y_hbm.at[i1_v], b1_v, sem)
    c0.wait()
    c1.wait()

    def row(i, _):
        for j in range(D // L):
            sl = pl.ds(j * L, L)
            b0_v[i, sl] = b0_v[i, sl] + b1_v[i, sl]
        return 0

    lax.fori_loop(0, _C_ROWS, row, 0)
    pltpu.sync_copy(b0_v, out_hbm.at[pl.ds(base, _C_ROWS)])


def _combine_sc(y, pos0, pos1):
    return pl.kernel(
        _combine_body,
        mesh=plsc.VectorSubcoreMesh(core_axis_name="c", subcore_axis_name="s"),
        out_type=jax.ShapeDtypeStruct((N, D), jnp.float32),
        scratch_types=[
            pltpu.VMEM((_C_ROWS,), jnp.int32),
            pltpu.VMEM((_C_ROWS,), jnp.int32),
            pltpu.VMEM((_C_ROWS, D), jnp.float32),
            pltpu.VMEM((_C_ROWS, D), jnp.float32),
            pltpu.SemaphoreType.DMA,
        ],
    )(y, pos0, pos1)


def kernel(x, gate_w, gate_b, w1, b1, w2, b2, wp, bp):
    x_flat = x.reshape(N, D)
    pos, probs, be, loss = _router(x_flat, gate_w, gate_b)
    pr0 = jnp.broadcast_to(probs[:, 0:1], (N, PW))
    pr1 = jnp.broadcast_to(probs[:, 1:2], (N, PW))
    xg, prows = _dispatch_sc(x_flat, pos[:, 0], pos[:, 1], pr0, pr1)
    y = _ffn(be.reshape(NB), xg, w1, b1, w2, b2, wp, bp, prows)
    out_flat = _combine_sc(y, pos[:, 0], pos[:, 1])
    return out_flat.reshape(B, T, D), loss.reshape(())
